# Initial kernel scaffold; baseline (speedup 1.0000x reference)
import jax
import jax.numpy as jnp
from jax.experimental import pallas as pl


def kernel(feats, segment_ids, W, b):
    raise NotImplementedError("write your pallas kernel here")


# speedup vs baseline: 2.4446x; 2.4446x over previous
import functools

import jax
import jax.numpy as jnp
from jax import lax
from jax.experimental import pallas as pl
from jax.experimental.pallas import tpu as pltpu
from jax.experimental.pallas import tpu_sc as plsc

N = 100000
D = 128
G = 1024
NC = 2
NS = 16
NW = NC * NS
GPW = G // NW
CH = 256
L = 16
NEG_INF = float("-inf")

_mesh = plsc.VectorSubcoreMesh(
    core_axis_name="c", subcore_axis_name="s", num_cores=NC, num_subcores=NS
)


@functools.partial(
    pl.kernel,
    out_type=jax.ShapeDtypeStruct((G, 2 * D), jnp.float32),
    mesh=_mesh,
    scratch_types=[
        pltpu.VMEM((CH, D), jnp.float32),
        pltpu.VMEM((G + L,), jnp.int32),
        pltpu.VMEM((D,), jnp.float32),
        pltpu.VMEM((L,), jnp.float32),
        pltpu.VMEM((GPW, 2 * D), jnp.float32),
    ],
)
def _readout(feats_hbm, starts_hbm, w_hbm, b_hbm, out_hbm,
             fbuf, starts_v, wbuf, bbuf, obuf):
    wid = lax.axis_index("s") * NC + lax.axis_index("c")
    g0 = wid * GPW

    def sread(idx):
        return starts_v[pl.ds(idx, L)][0]

    pltpu.sync_copy(starts_hbm, starts_v)
    pltpu.sync_copy(w_hbm, wbuf)
    pltpu.sync_copy(b_hbm, bbuf)

    wv = tuple(wbuf[pl.ds(j * L, L)] for j in range(D // L))
    bv = bbuf[...]
    zeros = jnp.zeros((L,), jnp.float32)
    ninf = jnp.full((L,), NEG_INF, jnp.float32)

    def init_row(r, _):
        for j in range(D // L):
            obuf[r, pl.ds(j * L, L)] = zeros
            obuf[r, pl.ds(D + j * L, L)] = ninf
        return 0
    lax.fori_loop(0, GPW, init_row, 0)

    s_w = sread(g0)
    e_w = sread(g0 + GPW)
    a_w = (s_w // 8) * 8
    nchunks = (e_w - a_w + (CH - 1)) // CH

    def chunk_body(ci, _):
        raw = a_w + ci * CH
        base = pl.multiple_of(jnp.minimum(raw, N - CH), 8)
        lo_c = jnp.maximum(s_w, raw)
        hi_c = jnp.minimum(raw + CH, e_w)
        pltpu.sync_copy(feats_hbm.at[pl.ds(base, CH)], fbuf)

        def graph_body(g, _):
            sg = sread(g0 + g)
            eg = sread(g0 + g + 1)
            lo = jnp.maximum(sg, lo_c)
            hi = jnp.minimum(eg, hi_c)

            @pl.when(hi > lo)
            def _():
                def node_body(i, acc):
                    off = i - base
                    f = tuple(fbuf[off, pl.ds(j * L, L)] for j in range(D // L))
                    p = [f[j] * wv[j] for j in range(D // L)]
                    t0 = (p[0] + p[1]) + (p[2] + p[3])
                    t1 = (p[4] + p[5]) + (p[6] + p[7])
                    t = t0 + t1
                    for m in (8, 4, 2, 1):
                        idx = jnp.arange(L, dtype=jnp.int32) ^ m
                        t = t + t.at[idx].get(mode="promise_in_bounds")
                    z = t + bv
                    gate = 1.0 / (1.0 + jnp.exp(-z))
                    new = []
                    for j in range(D // L):
                        new.append(acc[j] + f[j] * gate)
                    for j in range(D // L):
                        new.append(jnp.maximum(acc[D // L + j], f[j]))
                    return tuple(new)

                init = (zeros,) * (D // L) + (ninf,) * (D // L)
                fin = lax.fori_loop(lo, hi, node_body, init)
                for j in range(D // L):
                    sl = pl.ds(j * L, L)
                    obuf[g, sl] = obuf[g, sl] + fin[j]
                    sl2 = pl.ds(D + j * L, L)
                    obuf[g, sl2] = jnp.maximum(obuf[g, sl2], fin[D // L + j])
            return 0

        lax.fori_loop(0, GPW, graph_body, 0)
        return 0

    lax.fori_loop(0, nchunks, chunk_body, 0)
    pltpu.sync_copy(obuf, out_hbm.at[pl.ds(pl.multiple_of(g0, GPW), GPW)])


@jax.jit
def kernel(feats, segment_ids, W, b):
    seg32 = segment_ids.astype(jnp.int32)
    starts = jnp.searchsorted(
        seg32, jnp.arange(G + 1, dtype=jnp.int32), side="left"
    ).astype(jnp.int32)
    starts = jnp.concatenate([starts, jnp.zeros((L - 1,), jnp.int32)])
    w_flat = W.reshape(D).astype(jnp.float32)
    b_vec = jnp.broadcast_to(b.astype(jnp.float32), (L,))
    return _readout(feats, starts, w_flat, b_vec)


# speedup vs baseline: 2.6088x; 1.0672x over previous
import functools

import jax
import jax.numpy as jnp
from jax import lax
from jax.experimental import pallas as pl
from jax.experimental.pallas import tpu as pltpu
from jax.experimental.pallas import tpu_sc as plsc

N = 100000
D = 128
G = 1024
NC = 2
NS = 16
NW = NC * NS
GPW = G // NW
CH = 256
NB = 2
L = 16
JG = D // L
NEG_INF = float("-inf")

_mesh = plsc.VectorSubcoreMesh(
    core_axis_name="c", subcore_axis_name="s", num_cores=NC, num_subcores=NS
)


@functools.partial(
    pl.kernel,
    out_type=jax.ShapeDtypeStruct((G, 2 * D), jnp.float32),
    mesh=_mesh,
    scratch_types=[
        pltpu.VMEM((NB, CH, D), jnp.float32),
        pltpu.VMEM((G + L,), jnp.int32),
        pltpu.VMEM((D,), jnp.float32),
        pltpu.VMEM((L,), jnp.float32),
        pltpu.VMEM((GPW, 2 * D), jnp.float32),
        pltpu.SemaphoreType.DMA((NB,)),
    ],
)
def _readout(feats_hbm, starts_hbm, w_hbm, b_hbm, out_hbm,
             fbuf, starts_v, wbuf, bbuf, obuf, sems):
    wid = lax.axis_index("s") * NC + lax.axis_index("c")
    g0 = wid * GPW

    def sread(idx):
        return starts_v[pl.ds(idx, L)][0]

    pltpu.sync_copy(starts_hbm, starts_v)
    pltpu.sync_copy(w_hbm, wbuf)
    pltpu.sync_copy(b_hbm, bbuf)

    wv = tuple(wbuf[pl.ds(j * L, L)] for j in range(JG))
    bv = bbuf[...]
    zeros = jnp.zeros((L,), jnp.float32)
    ninf = jnp.full((L,), NEG_INF, jnp.float32)

    def init_row(r, _):
        for j in range(JG):
            obuf[r, pl.ds(j * L, L)] = zeros
            obuf[r, pl.ds(D + j * L, L)] = ninf
        return 0
    lax.fori_loop(0, GPW, init_row, 0)

    s_w = sread(g0)
    e_w = sread(g0 + GPW)
    a_w = (s_w // 8) * 8
    nchunks = (e_w - a_w + (CH - 1)) // CH

    def chunk_base(ci):
        raw = a_w + ci * CH
        return raw, pl.multiple_of(jnp.minimum(raw, N - CH), 8)

    def dma(ci, b):
        _, base = chunk_base(ci)
        return pltpu.make_async_copy(
            feats_hbm.at[pl.ds(base, CH)], fbuf.at[b], sems.at[b]
        )

    @pl.when(nchunks > 0)
    def _():
        dma(0, 0).start()

    @pl.when(nchunks > 1)
    def _():
        dma(1, 1).start()

    def gate_of(f):
        p = [f[j] * wv[j] for j in range(JG)]
        t = ((p[0] + p[1]) + (p[2] + p[3])) + ((p[4] + p[5]) + (p[6] + p[7]))
        for m in (8, 4, 2, 1):
            idx = jnp.arange(L, dtype=jnp.int32) ^ m
            t = t + t.at[idx].get(mode="promise_in_bounds")
        z = t + bv
        return 1.0 / (1.0 + jnp.exp(-z))

    def chunk_body(ci, _):
        b = lax.rem(ci, NB)
        raw, base = chunk_base(ci)
        lo_c = jnp.maximum(s_w, raw)
        hi_c = jnp.minimum(raw + CH, e_w)
        dma(ci, b).wait()

        def load_row(off):
            return tuple(fbuf[b, off, pl.ds(j * L, L)] for j in range(JG))

        def graph_body(g, _):
            sg = sread(g0 + g)
            eg = sread(g0 + g + 1)
            lo = jnp.maximum(sg, lo_c)
            hi = jnp.minimum(eg, hi_c)

            @pl.when(hi > lo)
            def _():
                cnt = hi - lo
                npairs = cnt // 2

                def pair_body(k, acc):
                    off = (lo - base) + 2 * k
                    f0 = load_row(off)
                    f1 = load_row(off + 1)
                    gate0 = gate_of(f0)
                    gate1 = gate_of(f1)
                    new = []
                    for j in range(JG):
                        new.append(acc[j] + (f0[j] * gate0 + f1[j] * gate1))
                    for j in range(JG):
                        new.append(
                            jnp.maximum(acc[JG + j], jnp.maximum(f0[j], f1[j]))
                        )
                    return tuple(new)

                def node_body(i, acc):
                    f = load_row(i - base)
                    gate = gate_of(f)
                    new = []
                    for j in range(JG):
                        new.append(acc[j] + f[j] * gate)
                    for j in range(JG):
                        new.append(jnp.maximum(acc[JG + j], f[j]))
                    return tuple(new)

                init = (zeros,) * JG + (ninf,) * JG
                fin = lax.fori_loop(0, npairs, pair_body, init)
                fin = lax.fori_loop(lo + 2 * npairs, hi, node_body, fin)
                for j in range(JG):
                    sl = pl.ds(j * L, L)
                    obuf[g, sl] = obuf[g, sl] + fin[j]
                    sl2 = pl.ds(D + j * L, L)
                    obuf[g, sl2] = jnp.maximum(obuf[g, sl2], fin[JG + j])
            return 0

        lax.fori_loop(0, GPW, graph_body, 0)

        @pl.when(ci + NB < nchunks)
        def _():
            dma(ci + NB, b).start()
        return 0

    lax.fori_loop(0, nchunks, chunk_body, 0)
    pltpu.sync_copy(obuf, out_hbm.at[pl.ds(pl.multiple_of(g0, GPW), GPW)])


@jax.jit
def kernel(feats, segment_ids, W, b):
    seg32 = segment_ids.astype(jnp.int32)
    starts = jnp.searchsorted(
        seg32, jnp.arange(G + 1, dtype=jnp.int32), side="left"
    ).astype(jnp.int32)
    starts = jnp.concatenate([starts, jnp.zeros((L - 1,), jnp.int32)])
    w_flat = W.reshape(D).astype(jnp.float32)
    b_vec = jnp.broadcast_to(b.astype(jnp.float32), (L,))
    return _readout(feats, starts, w_flat, b_vec)


# speedup vs baseline: 2.7650x; 1.0599x over previous
import functools

import jax
import jax.numpy as jnp
from jax import lax
from jax.experimental import pallas as pl
from jax.experimental.pallas import tpu as pltpu
from jax.experimental.pallas import tpu_sc as plsc

N = 100000
D = 128
G = 1024
NC = 2
NS = 16
NW = NC * NS
GPW = G // NW
CH = 256
NB = 2
L = 16
JG = D // L
NEG_INF = float("-inf")

_mesh = plsc.VectorSubcoreMesh(
    core_axis_name="c", subcore_axis_name="s", num_cores=NC, num_subcores=NS
)


@functools.partial(
    pl.kernel,
    out_type=jax.ShapeDtypeStruct((G, 2 * D), jnp.float32),
    mesh=_mesh,
    scratch_types=[
        pltpu.VMEM((NB, CH, D), jnp.float32),
        pltpu.VMEM((G + L,), jnp.int32),
        pltpu.VMEM((D,), jnp.float32),
        pltpu.VMEM((L,), jnp.float32),
        pltpu.VMEM((GPW, 2 * D), jnp.float32),
        pltpu.SemaphoreType.DMA((NB,)),
    ],
)
def _readout(feats_hbm, starts_hbm, w_hbm, b_hbm, out_hbm,
             fbuf, starts_v, wbuf, bbuf, obuf, sems):
    wid = lax.axis_index("s") * NC + lax.axis_index("c")
    g0 = wid * GPW

    def sread(idx):
        return starts_v[pl.ds(idx, L)][0]

    pltpu.sync_copy(starts_hbm, starts_v)
    pltpu.sync_copy(w_hbm, wbuf)
    pltpu.sync_copy(b_hbm, bbuf)

    wv = tuple(wbuf[pl.ds(j * L, L)] for j in range(JG))
    bv = bbuf[...]
    zeros = jnp.zeros((L,), jnp.float32)
    ninf = jnp.full((L,), NEG_INF, jnp.float32)

    def init_row(r, _):
        for j in range(JG):
            obuf[r, pl.ds(j * L, L)] = zeros
            obuf[r, pl.ds(D + j * L, L)] = ninf
        return 0
    lax.fori_loop(0, GPW, init_row, 0)

    s_w = sread(g0)
    e_w = sread(g0 + GPW)
    a_w = (s_w // 8) * 8
    nchunks = (e_w - a_w + (CH - 1)) // CH

    def chunk_base(ci):
        raw = a_w + ci * CH
        return raw, pl.multiple_of(jnp.minimum(raw, N - CH), 8)

    def dma(ci, b):
        _, base = chunk_base(ci)
        return pltpu.make_async_copy(
            feats_hbm.at[pl.ds(base, CH)], fbuf.at[b], sems.at[b]
        )

    @pl.when(nchunks > 0)
    def _():
        dma(0, 0).start()

    @pl.when(nchunks > 1)
    def _():
        dma(1, 1).start()

    def gate_of(f):
        p = [f[j] * wv[j] for j in range(JG)]
        t = ((p[0] + p[1]) + (p[2] + p[3])) + ((p[4] + p[5]) + (p[6] + p[7]))
        for m in (8, 4, 2, 1):
            idx = jnp.arange(L, dtype=jnp.int32) ^ m
            t = t + t.at[idx].get(mode="promise_in_bounds")
        z = t + bv
        return 1.0 / (1.0 + jnp.exp(-z))

    def chunk_body(ci, _):
        b = lax.rem(ci, NB)
        raw, base = chunk_base(ci)
        lo_c = jnp.maximum(s_w, raw)
        hi_c = jnp.minimum(raw + CH, e_w)
        dma(ci, b).wait()

        def load_row(off):
            return tuple(fbuf[b, off, pl.ds(j * L, L)] for j in range(JG))

        def graph_body(g, _):
            sg = sread(g0 + g)
            eg = sread(g0 + g + 1)
            lo = jnp.maximum(sg, lo_c)
            hi = jnp.minimum(eg, hi_c)

            @pl.when(hi > lo)
            def _():
                cnt = hi - lo
                npairs = cnt // 2

                def pair_body(k, acc):
                    off = (lo - base) + 2 * k
                    f0 = load_row(off)
                    f1 = load_row(off + 1)
                    gate0 = gate_of(f0)
                    gate1 = gate_of(f1)
                    new = []
                    for j in range(JG):
                        new.append(acc[j] + (f0[j] * gate0 + f1[j] * gate1))
                    for j in range(JG):
                        new.append(
                            jnp.maximum(acc[JG + j], jnp.maximum(f0[j], f1[j]))
                        )
                    return tuple(new)

                def node_body(i, acc):
                    f = load_row(i - base)
                    gate = gate_of(f)
                    new = []
                    for j in range(JG):
                        new.append(acc[j] + f[j] * gate)
                    for j in range(JG):
                        new.append(jnp.maximum(acc[JG + j], f[j]))
                    return tuple(new)

                init = (zeros,) * JG + (ninf,) * JG
                fin = lax.fori_loop(0, npairs, pair_body, init)
                fin = lax.fori_loop(lo + 2 * npairs, hi, node_body, fin)
                for j in range(JG):
                    sl = pl.ds(j * L, L)
                    obuf[g, sl] = obuf[g, sl] + fin[j]
                    sl2 = pl.ds(D + j * L, L)
                    obuf[g, sl2] = jnp.maximum(obuf[g, sl2], fin[JG + j])
            return 0

        lax.fori_loop(0, GPW, graph_body, 0)

        @pl.when(ci + NB < nchunks)
        def _():
            dma(ci + NB, b).start()
        return 0

    lax.fori_loop(0, nchunks, chunk_body, 0)
    pltpu.sync_copy(obuf, out_hbm.at[pl.ds(pl.multiple_of(g0, GPW), GPW)])


@jax.jit
def kernel(feats, segment_ids, W, b):
    seg32 = segment_ids.astype(jnp.int32)
    starts = jnp.searchsorted(
        seg32, jnp.arange(G + 1, dtype=jnp.int32), side="left",
        method="scan_unrolled",
    ).astype(jnp.int32)
    starts = jnp.concatenate([starts, jnp.zeros((L - 1,), jnp.int32)])
    w_flat = W.reshape(D).astype(jnp.float32)
    b_vec = jnp.broadcast_to(b.astype(jnp.float32), (L,))
    return _readout(feats, starts, w_flat, b_vec)


# speedup vs baseline: 5.2867x; 1.9120x over previous
import functools

import jax
import jax.numpy as jnp
from jax import lax
from jax.experimental import pallas as pl
from jax.experimental.pallas import tpu as pltpu
from jax.experimental.pallas import tpu_sc as plsc

N = 100000
D = 128
G = 1024
NC = 2
NS = 16
NW = NC * NS
GPW = G // NW
CH = 256
NB = 2
L = 16
JG = D // L
NV = (G + L) // L
NGRP = N // L
GPT = (NGRP + NS - 1) // NS
SCHG = 64
NCH0 = (GPT + SCHG - 1) // SCHG
SLEN = SCHG * L + L
NEG_INF = float("-inf")

_mesh = plsc.VectorSubcoreMesh(
    core_axis_name="c", subcore_axis_name="s", num_cores=NC, num_subcores=NS
)


@functools.partial(
    pl.kernel,
    out_type=jax.ShapeDtypeStruct((G, 2 * D), jnp.float32),
    mesh=_mesh,
    scratch_types=[
        pltpu.VMEM((NB, CH, D), jnp.float32),
        pltpu.VMEM((SLEN,), jnp.int32),
        pltpu.VMEM((G + L,), jnp.int32),
        pltpu.VMEM((G + L,), jnp.int32),
        pltpu.VMEM_SHARED((NS, G + L), jnp.int32),
        pltpu.VMEM((D,), jnp.float32),
        pltpu.VMEM((L,), jnp.float32),
        pltpu.VMEM((GPW, 2 * D), jnp.float32),
        pltpu.SemaphoreType.DMA((NB,)),
    ],
    compiler_params=pltpu.CompilerParams(needs_layout_passes=False),
)
def _readout(feats_hbm, seg_hbm, w_hbm, b_hbm, out_hbm,
             fbuf, sbuf, starts_v, tmpv, slab, wbuf, bbuf, obuf, sems):
    cid = lax.axis_index("c")
    sid = lax.axis_index("s")
    wid = sid * NC + cid
    g0 = wid * GPW
    iota = jnp.arange(L, dtype=jnp.int32)

    pltpu.sync_copy(w_hbm, wbuf)
    pltpu.sync_copy(b_hbm, bbuf)

    bigv = jnp.full((L,), N, jnp.int32)

    def init_cand(k, _):
        starts_v[pl.ds(k * L, L)] = bigv
        return 0
    lax.fori_loop(0, NV, init_cand, 0)

    g_lo = sid * GPT
    g_hi = jnp.minimum(g_lo + GPT, NGRP)

    def ch0_body(c, _):
        glo_c = g_lo + c * SCHG
        ghi_c = jnp.minimum(glo_c + SCHG, g_hi)

        @pl.when(ghi_c > glo_c)
        def _():
            nb = glo_c * L - L
            bb = jnp.clip(nb, 0, N - SLEN)
            pltpu.sync_copy(seg_hbm.at[pl.ds(pl.multiple_of(bb, 8), SLEN)], sbuf)

            def grp_body(gi, _):
                goff = gi * L - bb
                ids = sbuf[pl.ds(goff, L)]
                prev_mem = sbuf[pl.ds(jnp.maximum(goff - 1, 0), L)]
                prev_reg = ids.at[jnp.maximum(iota - 1, 0)].get(
                    mode="promise_in_bounds")
                prev = jnp.where(goff == 0, prev_reg, prev_mem)
                gidx = gi * L + iota
                mask = (ids != prev) | (gidx == 0)
                plsc.store_scatter(starts_v, [ids], gidx, mask=mask)
                return 0
            lax.fori_loop(glo_c, ghi_c, grp_body, 0)
        return 0
    lax.fori_loop(0, NCH0, ch0_body, 0)

    pltpu.sync_copy(starts_v, slab.at[sid])
    plsc.subcore_barrier()

    def merge_row(r, _):
        pltpu.sync_copy(slab.at[r], tmpv)

        def mr(k, _):
            sl = pl.ds(k * L, L)
            starts_v[sl] = jnp.minimum(starts_v[sl], tmpv[sl])
            return 0
        lax.fori_loop(0, NV, mr, 0)
        return 0
    lax.fori_loop(0, NS, merge_row, 0)

    def suf_body(k, carry):
        k2 = NV - 1 - k
        sl = pl.ds(k2 * L, L)
        v = starts_v[sl]
        for sh in (1, 2, 4, 8):
            idx = jnp.minimum(iota + sh, L - 1)
            shifted = v.at[idx].get(mode="promise_in_bounds")
            shifted = jnp.where(iota + sh > L - 1, N, shifted)
            v = jnp.minimum(v, shifted)
        v = jnp.minimum(v, carry)
        starts_v[sl] = v
        return v.at[jnp.zeros((L,), jnp.int32)].get(mode="promise_in_bounds")
    lax.fori_loop(0, NV, suf_body, bigv)

    def sread(idx):
        return starts_v[pl.ds(idx, L)][0]

    wv = tuple(wbuf[pl.ds(j * L, L)] for j in range(JG))
    bv = bbuf[...]
    zeros = jnp.zeros((L,), jnp.float32)
    ninf = jnp.full((L,), NEG_INF, jnp.float32)

    def init_row(r, _):
        for j in range(JG):
            obuf[r, pl.ds(j * L, L)] = zeros
            obuf[r, pl.ds(D + j * L, L)] = ninf
        return 0
    lax.fori_loop(0, GPW, init_row, 0)

    s_w = sread(g0)
    e_w = sread(g0 + GPW)
    a_w = (s_w // 8) * 8
    nchunks = (e_w - a_w + (CH - 1)) // CH

    def chunk_base(ci):
        raw = a_w + ci * CH
        return raw, pl.multiple_of(jnp.minimum(raw, N - CH), 8)

    def dma(ci, b):
        _, base = chunk_base(ci)
        return pltpu.make_async_copy(
            feats_hbm.at[pl.ds(base, CH)], fbuf.at[b], sems.at[b]
        )

    @pl.when(nchunks > 0)
    def _():
        dma(0, 0).start()

    @pl.when(nchunks > 1)
    def _():
        dma(1, 1).start()

    def gate_of(f):
        p = [f[j] * wv[j] for j in range(JG)]
        t = ((p[0] + p[1]) + (p[2] + p[3])) + ((p[4] + p[5]) + (p[6] + p[7]))
        for m in (8, 4, 2, 1):
            t = t + t.at[iota ^ m].get(mode="promise_in_bounds")
        z = t + bv
        return 1.0 / (1.0 + jnp.exp(-z))

    def chunk_body(ci, _):
        b = lax.rem(ci, NB)
        raw, base = chunk_base(ci)
        lo_c = jnp.maximum(s_w, raw)
        hi_c = jnp.minimum(raw + CH, e_w)
        dma(ci, b).wait()

        def load_row(off):
            return tuple(fbuf[b, off, pl.ds(j * L, L)] for j in range(JG))

        def graph_body(g, _):
            sg = sread(g0 + g)
            eg = sread(g0 + g + 1)
            lo = jnp.maximum(sg, lo_c)
            hi = jnp.minimum(eg, hi_c)

            @pl.when(hi > lo)
            def _():
                cnt = hi - lo
                npairs = cnt // 2

                def pair_body(k, acc):
                    off = (lo - base) + 2 * k
                    f0 = load_row(off)
                    f1 = load_row(off + 1)
                    gate0 = gate_of(f0)
                    gate1 = gate_of(f1)
                    new = []
                    for j in range(JG):
                        new.append(acc[j] + (f0[j] * gate0 + f1[j] * gate1))
                    for j in range(JG):
                        new.append(
                            jnp.maximum(acc[JG + j], jnp.maximum(f0[j], f1[j]))
                        )
                    return tuple(new)

                def node_body(i, acc):
                    f = load_row(i - base)
                    gate = gate_of(f)
                    new = []
                    for j in range(JG):
                        new.append(acc[j] + f[j] * gate)
                    for j in range(JG):
                        new.append(jnp.maximum(acc[JG + j], f[j]))
                    return tuple(new)

                init = (zeros,) * JG + (ninf,) * JG
                fin = lax.fori_loop(0, npairs, pair_body, init)
                fin = lax.fori_loop(lo + 2 * npairs, hi, node_body, fin)
                for j in range(JG):
                    sl = pl.ds(j * L, L)
                    obuf[g, sl] = obuf[g, sl] + fin[j]
                    sl2 = pl.ds(D + j * L, L)
                    obuf[g, sl2] = jnp.maximum(obuf[g, sl2], fin[JG + j])
            return 0

        lax.fori_loop(0, GPW, graph_body, 0)

        @pl.when(ci + NB < nchunks)
        def _():
            dma(ci + NB, b).start()
        return 0

    lax.fori_loop(0, nchunks, chunk_body, 0)
    pltpu.sync_copy(obuf, out_hbm.at[pl.ds(pl.multiple_of(g0, GPW), GPW)])


@jax.jit
def kernel(feats, segment_ids, W, b):
    seg32 = segment_ids.astype(jnp.int32)
    w_flat = W.reshape(D).astype(jnp.float32)
    b_vec = jnp.broadcast_to(b.astype(jnp.float32), (L,))
    return _readout(feats, seg32, w_flat, b_vec)
